# Initial kernel scaffold; baseline (speedup 1.0000x reference)
import jax
import jax.numpy as jnp
from jax.experimental import pallas as pl


def kernel(x, pos_table):
    raise NotImplementedError("write your pallas kernel here")


# speedup vs baseline: 1.4602x; 1.4602x over previous
import jax
import jax.numpy as jnp
from jax.experimental import pallas as pl

_BS = 256


def _add_body(x_ref, pos_ref, out_ref):
    out_ref[0, :, :] = x_ref[0, :, :] + pos_ref[:, :]


def kernel(x, pos_table):
    batch, seq_len, d_model = x.shape
    nb = seq_len // _BS
    return pl.pallas_call(
        _add_body,
        grid=(nb, batch),
        in_specs=[
            pl.BlockSpec((1, _BS, d_model), lambda i, j: (j, i, 0)),
            pl.BlockSpec((_BS, d_model), lambda i, j: (i, 0)),
        ],
        out_specs=pl.BlockSpec((1, _BS, d_model), lambda i, j: (j, i, 0)),
        out_shape=jax.ShapeDtypeStruct(x.shape, x.dtype),
    )(x, pos_table[:seq_len])


# speedup vs baseline: 1.9317x; 1.3229x over previous
import jax
import jax.numpy as jnp
from jax.experimental import pallas as pl

_BS = 512


def _add_body(x_ref, pos_ref, out_ref):
    out_ref[0, :, :] = x_ref[0, :, :] + pos_ref[:, :]


def kernel(x, pos_table):
    batch, seq_len, d_model = x.shape
    nb = seq_len // _BS
    return pl.pallas_call(
        _add_body,
        grid=(nb, batch),
        in_specs=[
            pl.BlockSpec((1, _BS, d_model), lambda i, j: (j, i, 0)),
            pl.BlockSpec((_BS, d_model), lambda i, j: (i, 0)),
        ],
        out_specs=pl.BlockSpec((1, _BS, d_model), lambda i, j: (j, i, 0)),
        out_shape=jax.ShapeDtypeStruct(x.shape, x.dtype),
    )(x, pos_table[:seq_len])


# speedup vs baseline: 2.1132x; 1.0939x over previous
import jax
import jax.numpy as jnp
from jax.experimental import pallas as pl

_BS = 1024


def _add_body(x_ref, pos_ref, out_ref):
    out_ref[0, :, :] = x_ref[0, :, :] + pos_ref[:, :]


def kernel(x, pos_table):
    batch, seq_len, d_model = x.shape
    nb = seq_len // _BS
    return pl.pallas_call(
        _add_body,
        grid=(nb, batch),
        in_specs=[
            pl.BlockSpec((1, _BS, d_model), lambda i, j: (j, i, 0)),
            pl.BlockSpec((_BS, d_model), lambda i, j: (i, 0)),
        ],
        out_specs=pl.BlockSpec((1, _BS, d_model), lambda i, j: (j, i, 0)),
        out_shape=jax.ShapeDtypeStruct(x.shape, x.dtype),
    )(x, pos_table[:seq_len])


# speedup vs baseline: 2.2814x; 1.0796x over previous
import jax
import jax.numpy as jnp
from jax.experimental import pallas as pl

_BS = 2048


def _add_body(x_ref, pos_ref, out_ref):
    out_ref[0, :, :] = x_ref[0, :, :] + pos_ref[:, :]


def kernel(x, pos_table):
    batch, seq_len, d_model = x.shape
    nb = seq_len // _BS
    return pl.pallas_call(
        _add_body,
        grid=(nb, batch),
        in_specs=[
            pl.BlockSpec((1, _BS, d_model), lambda i, j: (j, i, 0)),
            pl.BlockSpec((_BS, d_model), lambda i, j: (i, 0)),
        ],
        out_specs=pl.BlockSpec((1, _BS, d_model), lambda i, j: (j, i, 0)),
        out_shape=jax.ShapeDtypeStruct(x.shape, x.dtype),
    )(x, pos_table[:seq_len])
